# baseline (speedup 1.0000x reference)
import jax
import jax.numpy as jnp
from jax.experimental import pallas as pl
from jax.experimental.pallas import tpu as pltpu

N = 10000
D_IN = 128
D_OUT = 128
CH = 400
NBUF = 3
NCHUNK = N // CH


def _manual_kernel(x_ref, w_ref, b_ref, adj_hbm, out_ref, buf, sem):
    def start_copy(c):
        j = jax.lax.rem(c, NBUF)
        pltpu.make_async_copy(
            adj_hbm.at[pl.ds(c * CH, CH), :],
            buf.at[j],
            sem.at[j],
        ).start()

    for c in range(NBUF):
        start_copy(c)

    def body(c, _):
        j = jax.lax.rem(c, NBUF)
        pltpu.make_async_copy(
            adj_hbm.at[pl.ds(c * CH, CH), :],
            buf.at[j],
            sem.at[j],
        ).wait()
        t = jnp.dot(buf[j], x_ref[...], preferred_element_type=jnp.float32)
        out_ref[pl.ds(c * CH, CH), :] = (
            jnp.dot(t, w_ref[...], preferred_element_type=jnp.float32)
            + b_ref[...]
        )

        @pl.when(c + NBUF < NCHUNK)
        def _():
            start_copy(c + NBUF)

        return ()

    jax.lax.fori_loop(0, NCHUNK, body, ())


@jax.jit
def kernel(input, adj, W, b):
    b2 = b.reshape(1, D_OUT)
    return pl.pallas_call(
        _manual_kernel,
        in_specs=[
            pl.BlockSpec((N, D_IN), lambda: (0, 0)),
            pl.BlockSpec((D_IN, D_OUT), lambda: (0, 0)),
            pl.BlockSpec((1, D_OUT), lambda: (0, 0)),
            pl.BlockSpec(memory_space=pltpu.MemorySpace.HBM),
        ],
        out_specs=pl.BlockSpec((N, D_OUT), lambda: (0, 0)),
        out_shape=jax.ShapeDtypeStruct((N, D_OUT), jnp.float32),
        scratch_shapes=[
            pltpu.VMEM((NBUF, CH, N), jnp.float32),
            pltpu.SemaphoreType.DMA((NBUF,)),
        ],
    )(input, W, b2, adj)


# speedup vs baseline: 1.0315x; 1.0315x over previous
import jax
import jax.numpy as jnp
from jax.experimental import pallas as pl

N = 10000
D_IN = 128
D_OUT = 128
BM = 400


def _fused_kernel(x_ref, w_ref, b_ref, adj_ref, out_ref):
    t = jnp.dot(adj_ref[...], x_ref[...], preferred_element_type=jnp.float32)
    out_ref[...] = (
        jnp.dot(t, w_ref[...], preferred_element_type=jnp.float32) + b_ref[...]
    )


@jax.jit
def kernel(input, adj, W, b):
    b2 = b.reshape(1, D_OUT)
    grid = (N // BM,)
    return pl.pallas_call(
        _fused_kernel,
        grid=grid,
        in_specs=[
            pl.BlockSpec((N, D_IN), lambda i: (0, 0)),
            pl.BlockSpec((D_IN, D_OUT), lambda i: (0, 0)),
            pl.BlockSpec((1, D_OUT), lambda i: (0, 0)),
            pl.BlockSpec((BM, N), lambda i: (i, 0)),
        ],
        out_specs=pl.BlockSpec((BM, D_OUT), lambda i: (i, 0)),
        out_shape=jax.ShapeDtypeStruct((N, D_OUT), jnp.float32),
    )(input, W, b2, adj)
